# baseline (speedup 1.0000x reference)
import jax
import jax.numpy as jnp
from jax.experimental import pallas as pl
from jax.experimental.pallas import tpu as pltpu

_MEM = 100000
_D = 256
_B = 2000
_NB = _MEM // _B


def _argmax_body(ptr_ref, q_ref, keys_ref, idx_ref, best_v, best_i):
    i = pl.program_id(0)

    @pl.when(i == 0)
    def _():
        best_v[0] = -jnp.inf
        best_i[0] = 0

    q = q_ref[...]
    keys = keys_ref[...]
    qn = jnp.sqrt(jnp.sum(q * q))
    dots = jnp.sum(keys * q, axis=1)
    kn = jnp.sqrt(jnp.sum(keys * keys, axis=1))
    sim = dots / jnp.maximum(qn * kn, 1e-8)
    gidx = i * _B + jax.lax.iota(jnp.int32, _B)
    sim = jnp.where(gidx < ptr_ref[0], sim, -jnp.inf)
    m = jnp.max(sim)
    li = jnp.argmax(sim).astype(jnp.int32)

    @pl.when(m > best_v[0])
    def _():
        best_v[0] = m
        best_i[0] = i * _B + li

    @pl.when(i == pl.num_programs(0) - 1)
    def _():
        idx_ref[0] = best_i[0]


def _gather_body(idx_ref, mem_ref, out_ref):
    del idx_ref
    out_ref[...] = mem_ref[0, 1:2, :]


def kernel(query, memory, pointer):
    q2 = query.reshape(1, _D)
    ptr = jnp.asarray(pointer, jnp.int32).reshape(1)

    idx = pl.pallas_call(
        _argmax_body,
        grid_spec=pltpu.PrefetchScalarGridSpec(
            num_scalar_prefetch=1,
            grid=(_NB,),
            in_specs=[
                pl.BlockSpec((1, _D), lambda i, p: (0, 0)),
                pl.BlockSpec((_B, _D), lambda i, p: (i, 0)),
            ],
            out_specs=pl.BlockSpec(memory_space=pltpu.SMEM),
            scratch_shapes=[
                pltpu.SMEM((1,), jnp.float32),
                pltpu.SMEM((1,), jnp.int32),
            ],
        ),
        out_shape=jax.ShapeDtypeStruct((1,), jnp.int32),
    )(ptr, q2, memory)

    mem3 = memory.reshape(_MEM, 2, _D)
    row = pl.pallas_call(
        _gather_body,
        grid_spec=pltpu.PrefetchScalarGridSpec(
            num_scalar_prefetch=1,
            grid=(1,),
            in_specs=[
                pl.BlockSpec((1, 2, _D), lambda i, s: (s[0], 0, 0)),
            ],
            out_specs=pl.BlockSpec((1, _D), lambda i, s: (0, 0)),
        ),
        out_shape=jax.ShapeDtypeStruct((1, _D), jnp.float32),
    )(idx, mem3)

    return row.reshape(_D)
